# baseline (device time: 21949 ns/iter reference)
import jax
import jax.numpy as jnp
from jax import lax
from jax.experimental import pallas as pl
from jax.experimental.pallas import tpu as pltpu

B, H, W, C = 2, 64, 64, 64
NPIX_GLOBAL = (2 * H) * (2 * W)


def kernel(x, k, Wp):
    def body(x_ref, k_ref, wp_ref, out_ref,
             stats_send, stats_recv, row_send, col_send, cor_send,
             row_recv, col_recv, cor_recv, send_sems, recv_sems):
        px = lax.axis_index("x")
        py = lax.axis_index("y")
        x_nbr = (1 - px, py)
        y_nbr = (px, 1 - py)
        diag = (1 - px, 1 - py)

        barrier = pltpu.get_barrier_semaphore()
        for dev in (x_nbr, y_nbr, diag):
            pl.semaphore_signal(barrier, inc=1, device_id=dev,
                                device_id_type=pl.DeviceIdType.MESH)
        pl.semaphore_wait(barrier, 3)

        xv = x_ref[...]

        row_edge = jnp.where(px == 0, xv[:, H - 1, :, :], xv[:, 0, :, :])
        col_edge = jnp.where(py == 0, xv[:, :, W - 1, :], xv[:, :, 0, :])
        corner = jnp.where(py == 0, row_edge[:, W - 1, :], row_edge[:, 0, :])
        row_send[...] = row_edge
        col_send[...] = col_edge
        cor_send[...] = corner

        def start_rdma(i, src, dst, dev):
            r = pltpu.make_async_remote_copy(
                src_ref=src, dst_ref=dst,
                send_sem=send_sems.at[i], recv_sem=recv_sems.at[i],
                device_id=dev, device_id_type=pl.DeviceIdType.MESH,
            )
            r.start()
            return r

        halo_rdmas = [
            start_rdma(3, row_send, row_recv, x_nbr),
            start_rdma(4, col_send, col_recv, y_nbr),
            start_rdma(5, cor_send, cor_recv, diag),
        ]

        stats_send[0] = jnp.sum(xv, axis=(1, 2))
        stats_send[1] = jnp.sum(xv * xv, axis=(1, 2))
        stats_rdmas = [
            start_rdma(0, stats_send, stats_recv.at[0], x_nbr),
            start_rdma(1, stats_send, stats_recv.at[1], y_nbr),
            start_rdma(2, stats_send, stats_recv.at[2], diag),
        ]

        for r in halo_rdmas:
            r.wait()

        h = xv
        row_h = row_recv[...]
        col_h = col_recv[...]
        cor_h = cor_recv[...]

        top = jnp.where(px == 0, h[:, 0, :, :], row_h)
        bot = jnp.where(px == 0, row_h, h[:, H - 1, :, :])
        left = jnp.where(py == 0, h[:, :, 0, :], col_h)
        right = jnp.where(py == 0, col_h, h[:, :, W - 1, :])
        tl = jnp.where(px == 0, left[:, 0, :],
                       jnp.where(py == 0, top[:, 0, :], cor_h))
        tr = jnp.where(px == 0, right[:, 0, :],
                       jnp.where(py == 0, cor_h, top[:, -1, :]))
        bl = jnp.where(px == 1, left[:, -1, :],
                       jnp.where(py == 0, bot[:, 0, :], cor_h))
        br = jnp.where(px == 0,
                       jnp.where(py == 0, cor_h, bot[:, -1, :]),
                       right[:, -1, :])

        mid = jnp.concatenate(
            [left[:, :, None, :], h, right[:, :, None, :]], axis=2)
        toprow = jnp.concatenate(
            [tl[:, None, None, :], top[:, None, :, :], tr[:, None, None, :]],
            axis=2)
        botrow = jnp.concatenate(
            [bl[:, None, None, :], bot[:, None, :, :], br[:, None, None, :]],
            axis=2)
        padded = jnp.concatenate([toprow, mid, botrow], axis=1)

        kv = k_ref[...]
        conv_raw = jnp.zeros((B, H, W, C), jnp.float32)
        for di in range(3):
            for dj in range(3):
                conv_raw = (
                    conv_raw + padded[:, di:di + H, dj:dj + W, :] * kv[di, dj]
                )

        for r in stats_rdmas:
            r.wait()
        tot = stats_send[...] + stats_recv[0] + stats_recv[1] + stats_recv[2]
        mean = tot[0] / NPIX_GLOBAL
        var = tot[1] / NPIX_GLOBAL - mean * mean
        inv = lax.rsqrt(var + 1e-5)
        ksum = jnp.sum(kv, axis=(0, 1))
        shift = ((mean * ksum) * inv)[:, None, None, :]
        scale = inv[:, None, None, :]

        conv = conv_raw * scale - shift
        a = conv * jax.nn.sigmoid(conv)
        y = jnp.dot(a.reshape(-1, C), wp_ref[...],
                    preferred_element_type=jnp.float32)
        out_ref[...] = xv + y.reshape(B, H, W, C)

    return pl.pallas_call(
        body,
        out_shape=jax.ShapeDtypeStruct((B, H, W, C), jnp.float32),
        in_specs=[pl.BlockSpec(memory_space=pltpu.VMEM)] * 3,
        out_specs=pl.BlockSpec(memory_space=pltpu.VMEM),
        scratch_shapes=[
            pltpu.VMEM((2, B, C), jnp.float32),
            pltpu.VMEM((3, 2, B, C), jnp.float32),
            pltpu.VMEM((B, W, C), jnp.float32),
            pltpu.VMEM((B, H, C), jnp.float32),
            pltpu.VMEM((B, C), jnp.float32),
            pltpu.VMEM((B, W, C), jnp.float32),
            pltpu.VMEM((B, H, C), jnp.float32),
            pltpu.VMEM((B, C), jnp.float32),
            pltpu.SemaphoreType.DMA((6,)),
            pltpu.SemaphoreType.DMA((6,)),
        ],
        compiler_params=pltpu.CompilerParams(collective_id=0),
    )(x, k, Wp)


# device time: 18579 ns/iter; 1.1814x vs baseline; 1.1814x over previous
import jax
import jax.numpy as jnp
from jax import lax
from jax.experimental import pallas as pl
from jax.experimental.pallas import tpu as pltpu

B, H, W, C = 2, 64, 64, 64
NPIX_GLOBAL = (2 * H) * (2 * W)


def kernel(x, k, Wp):
    def body(x_ref, k_ref, wp_ref, out_ref,
             stats_send, stats_recv, row_send, col_send, cor_send,
             row_recv, col_recv, cor_recv, send_sems, recv_sems):
        px = lax.axis_index("x")
        py = lax.axis_index("y")
        x_nbr = (1 - px, py)
        y_nbr = (px, 1 - py)
        diag = (1 - px, 1 - py)

        barrier = pltpu.get_barrier_semaphore()
        for dev in (x_nbr, y_nbr, diag):
            pl.semaphore_signal(barrier, inc=1, device_id=dev,
                                device_id_type=pl.DeviceIdType.MESH)
        pl.semaphore_wait(barrier, 3)

        xv = x_ref[...]

        row_edge = jnp.where(px == 0, xv[:, H - 1, :, :], xv[:, 0, :, :])
        col_edge = jnp.where(py == 0, xv[:, :, W - 1, :], xv[:, :, 0, :])
        corner = jnp.where(py == 0, row_edge[:, W - 1, :], row_edge[:, 0, :])
        row_send[...] = row_edge
        col_send[...] = col_edge
        cor_send[...] = corner

        def start_rdma(i, src, dst, dev):
            r = pltpu.make_async_remote_copy(
                src_ref=src, dst_ref=dst,
                send_sem=send_sems.at[i], recv_sem=recv_sems.at[i],
                device_id=dev, device_id_type=pl.DeviceIdType.MESH,
            )
            r.start()
            return r

        stats_send[0] = jnp.sum(xv, axis=(1, 2))
        stats_send[1] = jnp.sum(xv * xv, axis=(1, 2))

        rdmas = [
            start_rdma(0, stats_send, stats_recv.at[0], x_nbr),
            start_rdma(1, stats_send, stats_recv.at[1], y_nbr),
            start_rdma(2, stats_send, stats_recv.at[2], diag),
            start_rdma(3, row_send, row_recv, x_nbr),
            start_rdma(4, col_send, col_recv, y_nbr),
            start_rdma(5, cor_send, cor_recv, diag),
        ]
        stats_rdmas = []
        for r in rdmas:
            r.wait()

        h = xv
        row_h = row_recv[...]
        col_h = col_recv[...]
        cor_h = cor_recv[...]

        top = jnp.where(px == 0, h[:, 0, :, :], row_h)
        bot = jnp.where(px == 0, row_h, h[:, H - 1, :, :])
        left = jnp.where(py == 0, h[:, :, 0, :], col_h)
        right = jnp.where(py == 0, col_h, h[:, :, W - 1, :])
        tl = jnp.where(px == 0, left[:, 0, :],
                       jnp.where(py == 0, top[:, 0, :], cor_h))
        tr = jnp.where(px == 0, right[:, 0, :],
                       jnp.where(py == 0, cor_h, top[:, -1, :]))
        bl = jnp.where(px == 1, left[:, -1, :],
                       jnp.where(py == 0, bot[:, 0, :], cor_h))
        br = jnp.where(px == 0,
                       jnp.where(py == 0, cor_h, bot[:, -1, :]),
                       right[:, -1, :])

        mid = jnp.concatenate(
            [left[:, :, None, :], h, right[:, :, None, :]], axis=2)
        toprow = jnp.concatenate(
            [tl[:, None, None, :], top[:, None, :, :], tr[:, None, None, :]],
            axis=2)
        botrow = jnp.concatenate(
            [bl[:, None, None, :], bot[:, None, :, :], br[:, None, None, :]],
            axis=2)
        padded = jnp.concatenate([toprow, mid, botrow], axis=1)

        kv = k_ref[...]
        conv_raw = jnp.zeros((B, H, W, C), jnp.float32)
        for di in range(3):
            for dj in range(3):
                conv_raw = (
                    conv_raw + padded[:, di:di + H, dj:dj + W, :] * kv[di, dj]
                )

        for r in stats_rdmas:
            r.wait()
        tot = stats_send[...] + stats_recv[0] + stats_recv[1] + stats_recv[2]
        mean = tot[0] / NPIX_GLOBAL
        var = tot[1] / NPIX_GLOBAL - mean * mean
        inv = lax.rsqrt(var + 1e-5)
        ksum = jnp.sum(kv, axis=(0, 1))
        shift = ((mean * ksum) * inv)[:, None, None, :]
        scale = inv[:, None, None, :]

        conv = conv_raw * scale - shift
        a = conv * jax.nn.sigmoid(conv)
        y = jnp.dot(a.reshape(-1, C), wp_ref[...],
                    preferred_element_type=jnp.float32)
        out_ref[...] = xv + y.reshape(B, H, W, C)

    return pl.pallas_call(
        body,
        out_shape=jax.ShapeDtypeStruct((B, H, W, C), jnp.float32),
        in_specs=[pl.BlockSpec(memory_space=pltpu.VMEM)] * 3,
        out_specs=pl.BlockSpec(memory_space=pltpu.VMEM),
        scratch_shapes=[
            pltpu.VMEM((2, B, C), jnp.float32),
            pltpu.VMEM((3, 2, B, C), jnp.float32),
            pltpu.VMEM((B, W, C), jnp.float32),
            pltpu.VMEM((B, H, C), jnp.float32),
            pltpu.VMEM((B, C), jnp.float32),
            pltpu.VMEM((B, W, C), jnp.float32),
            pltpu.VMEM((B, H, C), jnp.float32),
            pltpu.VMEM((B, C), jnp.float32),
            pltpu.SemaphoreType.DMA((6,)),
            pltpu.SemaphoreType.DMA((6,)),
        ],
        compiler_params=pltpu.CompilerParams(collective_id=0),
    )(x, k, Wp)


# device time: 18411 ns/iter; 1.1922x vs baseline; 1.0091x over previous
import jax
import jax.numpy as jnp
from jax import lax
from jax.experimental import pallas as pl
from jax.experimental.pallas import tpu as pltpu

B, H, W, C = 2, 64, 64, 64
NPIX_GLOBAL = (2 * H) * (2 * W)


def kernel(x, k, Wp):
    def body(x_ref, k_ref, wp_ref, out_ref,
             stats_send, stats_recv, row_send, col_send, cor_send,
             row_recv, col_recv, cor_recv, send_sems, recv_sems):
        px = lax.axis_index("x")
        py = lax.axis_index("y")
        x_nbr = (1 - px, py)
        y_nbr = (px, 1 - py)
        diag = (1 - px, 1 - py)

        barrier = pltpu.get_barrier_semaphore()
        for dev in (x_nbr, y_nbr, diag):
            pl.semaphore_signal(barrier, inc=1, device_id=dev,
                                device_id_type=pl.DeviceIdType.MESH)

        xv = x_ref[...]

        row_edge = jnp.where(px == 0, xv[:, H - 1, :, :], xv[:, 0, :, :])
        col_edge = jnp.where(py == 0, xv[:, :, W - 1, :], xv[:, :, 0, :])
        corner = jnp.where(py == 0, row_edge[:, W - 1, :], row_edge[:, 0, :])
        row_send[...] = row_edge
        col_send[...] = col_edge
        cor_send[...] = corner

        def start_rdma(i, src, dst, dev):
            r = pltpu.make_async_remote_copy(
                src_ref=src, dst_ref=dst,
                send_sem=send_sems.at[i], recv_sem=recv_sems.at[i],
                device_id=dev, device_id_type=pl.DeviceIdType.MESH,
            )
            r.start()
            return r

        stats_send[0] = jnp.sum(xv, axis=(1, 2))
        stats_send[1] = jnp.sum(xv * xv, axis=(1, 2))

        pl.semaphore_wait(barrier, 3)

        rdmas = [
            start_rdma(0, stats_send, stats_recv.at[0], x_nbr),
            start_rdma(1, stats_send, stats_recv.at[1], y_nbr),
            start_rdma(2, stats_send, stats_recv.at[2], diag),
            start_rdma(3, row_send, row_recv, x_nbr),
            start_rdma(4, col_send, col_recv, y_nbr),
            start_rdma(5, cor_send, cor_recv, diag),
        ]
        for r in rdmas:
            r.wait()

        h = xv
        row_h = row_recv[...]
        col_h = col_recv[...]
        cor_h = cor_recv[...]

        top = jnp.where(px == 0, h[:, 0, :, :], row_h)
        bot = jnp.where(px == 0, row_h, h[:, H - 1, :, :])
        left = jnp.where(py == 0, h[:, :, 0, :], col_h)
        right = jnp.where(py == 0, col_h, h[:, :, W - 1, :])
        tl = jnp.where(px == 0, left[:, 0, :],
                       jnp.where(py == 0, top[:, 0, :], cor_h))
        tr = jnp.where(px == 0, right[:, 0, :],
                       jnp.where(py == 0, cor_h, top[:, -1, :]))
        bl = jnp.where(px == 1, left[:, -1, :],
                       jnp.where(py == 0, bot[:, 0, :], cor_h))
        br = jnp.where(px == 0,
                       jnp.where(py == 0, cor_h, bot[:, -1, :]),
                       right[:, -1, :])

        mid = jnp.concatenate(
            [left[:, :, None, :], h, right[:, :, None, :]], axis=2)
        toprow = jnp.concatenate(
            [tl[:, None, None, :], top[:, None, :, :], tr[:, None, None, :]],
            axis=2)
        botrow = jnp.concatenate(
            [bl[:, None, None, :], bot[:, None, :, :], br[:, None, None, :]],
            axis=2)
        padded = jnp.concatenate([toprow, mid, botrow], axis=1)

        kv = k_ref[...]
        conv_raw = jnp.zeros((B, H, W, C), jnp.float32)
        for di in range(3):
            for dj in range(3):
                conv_raw = (
                    conv_raw + padded[:, di:di + H, dj:dj + W, :] * kv[di, dj]
                )

        tot = stats_send[...] + stats_recv[0] + stats_recv[1] + stats_recv[2]
        mean = tot[0] / NPIX_GLOBAL
        var = tot[1] / NPIX_GLOBAL - mean * mean
        inv = lax.rsqrt(var + 1e-5)
        ksum = jnp.sum(kv, axis=(0, 1))
        shift = ((mean * ksum) * inv)[:, None, None, :]
        scale = inv[:, None, None, :]

        conv = conv_raw * scale - shift
        a = conv * jax.nn.sigmoid(conv)
        y = jnp.dot(a.reshape(-1, C), wp_ref[...],
                    preferred_element_type=jnp.float32)
        out_ref[...] = xv + y.reshape(B, H, W, C)

    return pl.pallas_call(
        body,
        out_shape=jax.ShapeDtypeStruct((B, H, W, C), jnp.float32),
        in_specs=[pl.BlockSpec(memory_space=pltpu.VMEM)] * 3,
        out_specs=pl.BlockSpec(memory_space=pltpu.VMEM),
        scratch_shapes=[
            pltpu.VMEM((2, B, C), jnp.float32),
            pltpu.VMEM((3, 2, B, C), jnp.float32),
            pltpu.VMEM((B, W, C), jnp.float32),
            pltpu.VMEM((B, H, C), jnp.float32),
            pltpu.VMEM((B, C), jnp.float32),
            pltpu.VMEM((B, W, C), jnp.float32),
            pltpu.VMEM((B, H, C), jnp.float32),
            pltpu.VMEM((B, C), jnp.float32),
            pltpu.SemaphoreType.DMA((6,)),
            pltpu.SemaphoreType.DMA((6,)),
        ],
        compiler_params=pltpu.CompilerParams(collective_id=0),
    )(x, k, Wp)
